# baseline (speedup 1.0000x reference)
import functools

import jax
import jax.numpy as jnp
from jax import lax
from jax.experimental import pallas as pl
from jax.experimental.pallas import tpu as pltpu
from jax.experimental.pallas import tpu_sc as plsc

K = 1024
D = 64
BETA = 0.25
N = 8192
BLK = 512
NBLK = N // BLK

NC = 2
NS = 16
NW = NC * NS
RPW = N // NW
CH = 128
NCH = RPW // CH


def _dist_argmin_body(z_ref, cb_ref, idx_ref, mind_ref):
    z = z_ref[...]
    cb = cb_ref[...]
    s = lax.dot_general(z, cb, (((1,), (1,)), ((), ())),
                        preferred_element_type=jnp.float32)
    z2 = jnp.sum(z * z, axis=1, keepdims=True)
    c2 = jnp.sum(cb * cb, axis=1)
    d = z2 + c2 - 2.0 * s
    mind = jnp.min(d, axis=1)
    iota = lax.broadcasted_iota(jnp.int32, (BLK, K), 1)
    idx = jnp.min(jnp.where(d == mind[:, None], iota, K), axis=1)
    idx_ref[0, 0, :] = idx
    mind_ref[0, 0, :] = mind


def _tc_dist_argmin(z_flat, codebook):
    return pl.pallas_call(
        _dist_argmin_body,
        grid=(NBLK,),
        in_specs=[
            pl.BlockSpec((BLK, D), lambda i: (i, 0)),
            pl.BlockSpec((K, D), lambda i: (0, 0)),
        ],
        out_specs=[
            pl.BlockSpec((1, 1, BLK), lambda i: (i, 0, 0)),
            pl.BlockSpec((1, 1, BLK), lambda i: (i, 0, 0)),
        ],
        out_shape=[
            jax.ShapeDtypeStruct((NBLK, 1, BLK), jnp.int32),
            jax.ShapeDtypeStruct((NBLK, 1, BLK), jnp.float32),
        ],
    )(z_flat, codebook)


def _sc_gather_hist_body(cb_hbm, idx_hbm, zq_hbm, hist_hbm,
                         idx_v, rows_v, hist_v, tot_v, sem):
    wid = lax.axis_index("s") * NC + lax.axis_index("c")
    base = wid * RPW

    for c in range(NCH):
        pltpu.sync_copy(idx_hbm.at[pl.ds(base + c * CH, CH)], idx_v.at[c])
    copies = [pltpu.async_copy(cb_hbm.at[idx_v.at[c]], rows_v.at[c], sem)
              for c in range(NCH)]

    zeros16 = jnp.zeros((16,), jnp.float32)

    def zbody(j, carry):
        hist_v[pl.ds(j * 16, 16)] = zeros16
        return carry
    lax.fori_loop(0, 16 * K // 16, zbody, 0)

    lane_off = lax.iota(jnp.int32, 16) * K
    ones16 = jnp.ones((16,), jnp.float32)
    for c in range(NCH):
        for g in range(CH // 16):
            iv = idx_v[c, pl.ds(g * 16, 16)]
            plsc.addupdate_scatter(hist_v, [lane_off + iv], ones16)

    for cp in copies:
        cp.wait()
    for c in range(NCH):
        pltpu.sync_copy(rows_v.at[c], zq_hbm.at[pl.ds(base + c * CH, CH)])

    def rbody(j, carry):
        acc = hist_v[pl.ds(j * 16, 16)]
        for l in range(1, 16):
            acc = acc + hist_v[pl.ds(l * K + j * 16, 16)]
        tot_v[pl.ds(j * 16, 16)] = acc
        return carry
    lax.fori_loop(0, K // 16, rbody, 0)
    pltpu.sync_copy(tot_v, hist_hbm.at[wid])


def _sc_gather_hist(codebook, idx_flat):
    mesh = plsc.VectorSubcoreMesh(core_axis_name="c", subcore_axis_name="s")
    f = functools.partial(
        pl.kernel,
        mesh=mesh,
        out_type=[
            jax.ShapeDtypeStruct((N, D), jnp.float32),
            jax.ShapeDtypeStruct((NW, K), jnp.float32),
        ],
        scratch_types=[
            pltpu.VMEM((NCH, CH), jnp.int32),
            pltpu.VMEM((NCH, CH, D), jnp.float32),
            pltpu.VMEM((16 * K,), jnp.float32),
            pltpu.VMEM((K,), jnp.float32),
            pltpu.SemaphoreType.DMA,
        ],
        compiler_params=pltpu.CompilerParams(
            needs_layout_passes=False, use_tc_tiling_on_sc=False),
    )(_sc_gather_hist_body)
    return f(codebook, idx_flat)


def _finalize_body(mind_ref, hist_ref, loss_ref, perp_ref):
    total = jnp.sum(mind_ref[...])
    m = total / float(N * D)
    loss = m + BETA * m
    counts = jnp.sum(hist_ref[...], axis=0)
    e = counts * (1.0 / float(N))
    ent = jnp.sum(e * jnp.log(e + 1e-10))
    perp = jnp.exp(-ent)
    loss_ref[...] = jnp.full((1, 1), loss, jnp.float32)
    perp_ref[...] = jnp.full((1, 1), perp, jnp.float32)


def _tc_finalize(mind, hist):
    return pl.pallas_call(
        _finalize_body,
        out_shape=[
            jax.ShapeDtypeStruct((1, 1), jnp.float32),
            jax.ShapeDtypeStruct((1, 1), jnp.float32),
        ],
    )(mind, hist)


def kernel(z, codebook):
    z_flat = z.reshape(-1, D)
    idx3, mind = _tc_dist_argmin(z_flat, codebook)
    idx_flat = idx3.reshape(-1)
    zq_flat, hist = _sc_gather_hist(codebook, idx_flat)
    loss2, perp2 = _tc_finalize(mind, hist)
    z_q_st = zq_flat.reshape(z.shape)
    indices = idx_flat.reshape(z.shape[:-1])
    return (loss2[0, 0], z_q_st, perp2[0, 0], indices)


# speedup vs baseline: 1.2427x; 1.2427x over previous
import functools

import jax
import jax.numpy as jnp
from jax import lax
from jax.experimental import pallas as pl
from jax.experimental.pallas import tpu as pltpu
from jax.experimental.pallas import tpu_sc as plsc

K = 1024
D = 64
BETA = 0.25
N = 8192
BLK = 512
NBLK = N // BLK

NC = 1
NS = 16
NW = NC * NS
RPW = N // NW
CH = 128
NCH = RPW // CH


def _main_body(z_ref, cb_ref, idx_ref, loss_ref, perp_ref,
               c2_s, hist_s, lsum_s):
    i = pl.program_id(0)
    cb = cb_ref[...]

    @pl.when(i == 0)
    def _init():
        c2 = jnp.sum(cb * cb, axis=1)
        c2_s[...] = c2[None, :]
        hist_s[...] = jnp.zeros((1, K), jnp.float32)
        lsum_s[0] = 0.0

    z = z_ref[...]
    s = lax.dot_general(z, cb, (((1,), (1,)), ((), ())),
                        preferred_element_type=jnp.float32)
    z2 = jnp.sum(z * z, axis=1, keepdims=True)
    d = z2 + c2_s[...] - 2.0 * s
    mind = jnp.min(d, axis=1)
    iota1 = lax.broadcasted_iota(jnp.int32, (1, K), 1)
    idx = jnp.min(jnp.where(d == mind[:, None], iota1, K), axis=1)
    idx_ref[...] = idx
    lsum_s[0] += jnp.sum(mind)
    eqf = (idx[:, None] == iota1).astype(jnp.float32)
    ones_row = jnp.ones((1, BLK), jnp.float32)
    counts_blk = lax.dot_general(ones_row, eqf, (((1,), (0,)), ((), ())),
                                 preferred_element_type=jnp.float32)
    hist_s[...] += counts_blk

    @pl.when(i == NBLK - 1)
    def _fin():
        m = lsum_s[0] / float(N * D)
        counts = hist_s[...]
        e = counts * (1.0 / float(N))
        ent = jnp.sum(e * jnp.log(e + 1e-10))
        loss_ref[...] = jnp.full((1, 1), m + BETA * m, jnp.float32)
        perp_ref[...] = jnp.full((1, 1), jnp.exp(-ent), jnp.float32)


def _tc_main(z_flat, codebook):
    return pl.pallas_call(
        _main_body,
        grid=(NBLK,),
        in_specs=[
            pl.BlockSpec((BLK, D), lambda i: (i, 0)),
            pl.BlockSpec((K, D), lambda i: (0, 0)),
        ],
        out_specs=[
            pl.BlockSpec((BLK,), lambda i: (i,)),
            pl.BlockSpec((1, 1), lambda i: (0, 0)),
            pl.BlockSpec((1, 1), lambda i: (0, 0)),
        ],
        out_shape=[
            jax.ShapeDtypeStruct((N,), jnp.int32),
            jax.ShapeDtypeStruct((1, 1), jnp.float32),
            jax.ShapeDtypeStruct((1, 1), jnp.float32),
        ],
        scratch_shapes=[
            pltpu.VMEM((1, K), jnp.float32),
            pltpu.VMEM((1, K), jnp.float32),
            pltpu.SMEM((1,), jnp.float32),
        ],
    )(z_flat, codebook)


def _sc_gather_body(cb_hbm, idx_hbm, zq_hbm, idx_v, rows_v, sem):
    wid = lax.axis_index("s") * NC + lax.axis_index("c")
    base = wid * RPW
    for c in range(NCH):
        pltpu.sync_copy(idx_hbm.at[pl.ds(base + c * CH, CH)], idx_v.at[c])
    inflight = [None, None]
    for c in range(NCH):
        b = c % 2
        if inflight[b] is not None:
            cprev, bprev = inflight[b]
            cprev.wait()
            pltpu.sync_copy(rows_v.at[b], zq_hbm.at[pl.ds(base + bprev * CH, CH)])
        inflight[b] = (pltpu.async_copy(cb_hbm.at[idx_v.at[c]],
                                        rows_v.at[b], sem), c)
    for b in range(2):
        if inflight[b] is not None:
            cp, cdone = inflight[b]
            cp.wait()
            pltpu.sync_copy(rows_v.at[b], zq_hbm.at[pl.ds(base + cdone * CH, CH)])


def _sc_gather(codebook, idx_flat):
    mesh = plsc.VectorSubcoreMesh(core_axis_name="c", subcore_axis_name="s",
                                  num_cores=NC)
    f = functools.partial(
        pl.kernel,
        mesh=mesh,
        out_type=jax.ShapeDtypeStruct((N, D), jnp.float32),
        scratch_types=[
            pltpu.VMEM((NCH, CH), jnp.int32),
            pltpu.VMEM((2, CH, D), jnp.float32),
            pltpu.SemaphoreType.DMA,
        ],
        compiler_params=pltpu.CompilerParams(
            needs_layout_passes=False, use_tc_tiling_on_sc=False),
    )(_sc_gather_body)
    return f(codebook, idx_flat)


def kernel(z, codebook):
    z_flat = z.reshape(-1, D)
    idx_flat, loss2, perp2 = _tc_main(z_flat, codebook)
    zq_flat = _sc_gather(codebook, idx_flat)
    z_q_st = zq_flat.reshape(z.shape)
    indices = idx_flat.reshape(z.shape[:-1])
    return (loss2[0, 0], z_q_st, perp2[0, 0], indices)


# speedup vs baseline: 1.3610x; 1.0951x over previous
import functools

import jax
import jax.numpy as jnp
from jax import lax
from jax.experimental import pallas as pl
from jax.experimental.pallas import tpu as pltpu
from jax.experimental.pallas import tpu_sc as plsc

K = 1024
D = 64
BETA = 0.25
N = 8192
BLK = 4096
NBLK = N // BLK

NC = 2
NS = 16
NW = NC * NS
RPW = N // NW
CH = 128
NCH = RPW // CH


def _main_body(z_ref, cb_ref, idx_ref, loss_ref, perp_ref,
               c2_s, hist_s, lsum_s):
    i = pl.program_id(0)
    cb = cb_ref[...]

    @pl.when(i == 0)
    def _init():
        c2 = jnp.sum(cb * cb, axis=1)
        c2_s[...] = c2[None, :]
        hist_s[...] = jnp.zeros((1, K), jnp.float32)
        lsum_s[0] = 0.0

    z = z_ref[...]
    s = lax.dot_general(z, cb, (((1,), (1,)), ((), ())),
                        preferred_element_type=jnp.float32)
    z2 = jnp.sum(z * z, axis=1, keepdims=True)
    d = z2 + c2_s[...] - 2.0 * s
    mind = jnp.min(d, axis=1)
    iota1 = lax.broadcasted_iota(jnp.int32, (1, K), 1)
    idx = jnp.min(jnp.where(d == mind[:, None], iota1, K), axis=1)
    idx_ref[...] = idx
    lsum_s[0] += jnp.sum(mind)
    eqf = (idx[:, None] == iota1).astype(jnp.float32)
    ones_row = jnp.ones((1, BLK), jnp.float32)
    counts_blk = lax.dot_general(ones_row, eqf, (((1,), (0,)), ((), ())),
                                 preferred_element_type=jnp.float32)
    hist_s[...] += counts_blk

    @pl.when(i == NBLK - 1)
    def _fin():
        m = lsum_s[0] / float(N * D)
        counts = hist_s[...]
        e = counts * (1.0 / float(N))
        ent = jnp.sum(e * jnp.log(e + 1e-10))
        loss_ref[...] = jnp.full((1, 1), m + BETA * m, jnp.float32)
        perp_ref[...] = jnp.full((1, 1), jnp.exp(-ent), jnp.float32)


def _tc_main(z_flat, codebook):
    return pl.pallas_call(
        _main_body,
        grid=(NBLK,),
        in_specs=[
            pl.BlockSpec((BLK, D), lambda i: (i, 0)),
            pl.BlockSpec((K, D), lambda i: (0, 0)),
        ],
        out_specs=[
            pl.BlockSpec((BLK,), lambda i: (i,)),
            pl.BlockSpec((1, 1), lambda i: (0, 0)),
            pl.BlockSpec((1, 1), lambda i: (0, 0)),
        ],
        out_shape=[
            jax.ShapeDtypeStruct((N,), jnp.int32),
            jax.ShapeDtypeStruct((1, 1), jnp.float32),
            jax.ShapeDtypeStruct((1, 1), jnp.float32),
        ],
        scratch_shapes=[
            pltpu.VMEM((1, K), jnp.float32),
            pltpu.VMEM((1, K), jnp.float32),
            pltpu.SMEM((1,), jnp.float32),
        ],
    )(z_flat, codebook)


def _sc_gather_body(cb_hbm, idx_hbm, zq_hbm, idx_v, rows_v, sem):
    wid = lax.axis_index("s") * NC + lax.axis_index("c")
    base = wid * RPW
    for c in range(NCH):
        pltpu.sync_copy(idx_hbm.at[pl.ds(base + c * CH, CH)], idx_v.at[c])
    copies = [pltpu.async_copy(cb_hbm.at[idx_v.at[c]], rows_v.at[c], sem)
              for c in range(NCH)]
    for c in range(NCH):
        copies[c].wait()
        pltpu.sync_copy(rows_v.at[c], zq_hbm.at[pl.ds(base + c * CH, CH)])


def _sc_gather(codebook, idx_flat):
    mesh = plsc.VectorSubcoreMesh(core_axis_name="c", subcore_axis_name="s")
    f = functools.partial(
        pl.kernel,
        mesh=mesh,
        out_type=jax.ShapeDtypeStruct((N, D), jnp.float32),
        scratch_types=[
            pltpu.VMEM((NCH, CH), jnp.int32),
            pltpu.VMEM((NCH, CH, D), jnp.float32),
            pltpu.SemaphoreType.DMA,
        ],
        compiler_params=pltpu.CompilerParams(
            needs_layout_passes=False, use_tc_tiling_on_sc=False),
    )(_sc_gather_body)
    return f(codebook, idx_flat)


def kernel(z, codebook):
    z_flat = z.reshape(-1, D)
    idx_flat, loss2, perp2 = _tc_main(z_flat, codebook)
    zq_flat = _sc_gather(codebook, idx_flat)
    z_q_st = zq_flat.reshape(z.shape)
    indices = idx_flat.reshape(z.shape[:-1])
    return (loss2[0, 0], z_q_st, perp2[0, 0], indices)


# speedup vs baseline: 1.3625x; 1.0011x over previous
import functools

import jax
import jax.numpy as jnp
from jax import lax
from jax.experimental import pallas as pl
from jax.experimental.pallas import tpu as pltpu
from jax.experimental.pallas import tpu_sc as plsc

K = 1024
D = 64
BETA = 0.25
N = 8192
BLK = 4096
NBLK = N // BLK

NC = 2
NS = 16
NW = NC * NS
RPW = N // NW
CH = 128
NCH = RPW // CH


def _main_body(z_ref, cb_ref, idx_ref, loss_ref, perp_ref,
               c2_s, hist_s, lsum_s):
    i = pl.program_id(0)
    cb = cb_ref[...]

    @pl.when(i == 0)
    def _init():
        c2 = jnp.sum(cb * cb, axis=1)
        c2_s[...] = c2[None, :]
        hist_s[...] = jnp.zeros((1, K), jnp.float32)
        lsum_s[0] = 0.0

    z = z_ref[...]
    s = lax.dot_general(z, cb, (((1,), (1,)), ((), ())),
                        preferred_element_type=jnp.float32)
    z2 = jnp.sum(z * z, axis=1, keepdims=True)
    d = z2 + c2_s[...] - 2.0 * s
    mind = jnp.min(d, axis=1)
    iota1 = lax.broadcasted_iota(jnp.int32, (1, K), 1)
    idx = jnp.min(jnp.where(d == mind[:, None], iota1, K), axis=1)
    idx_ref[...] = idx
    lsum_s[0] += jnp.sum(mind)
    eqf = (idx[:, None] == iota1).astype(jnp.float32)
    ones_row = jnp.ones((1, BLK), jnp.float32)
    counts_blk = lax.dot_general(ones_row, eqf, (((1,), (0,)), ((), ())),
                                 preferred_element_type=jnp.float32)
    hist_s[...] += counts_blk

    @pl.when(i == NBLK - 1)
    def _fin():
        m = lsum_s[0] / float(N * D)
        counts = hist_s[...]
        e = counts * (1.0 / float(N))
        ent = jnp.sum(e * jnp.log(e + 1e-10))
        loss_ref[...] = jnp.full((1, 1), m + BETA * m, jnp.float32)
        perp_ref[...] = jnp.full((1, 1), jnp.exp(-ent), jnp.float32)


def _tc_main(z_flat, codebook):
    return pl.pallas_call(
        _main_body,
        grid=(NBLK,),
        in_specs=[
            pl.BlockSpec((BLK, D), lambda i: (i, 0)),
            pl.BlockSpec((K, D), lambda i: (0, 0)),
        ],
        out_specs=[
            pl.BlockSpec((BLK,), lambda i: (i,)),
            pl.BlockSpec((1, 1), lambda i: (0, 0)),
            pl.BlockSpec((1, 1), lambda i: (0, 0)),
        ],
        out_shape=[
            jax.ShapeDtypeStruct((N,), jnp.int32),
            jax.ShapeDtypeStruct((1, 1), jnp.float32),
            jax.ShapeDtypeStruct((1, 1), jnp.float32),
        ],
        scratch_shapes=[
            pltpu.VMEM((1, K), jnp.float32),
            pltpu.VMEM((1, K), jnp.float32),
            pltpu.SMEM((1,), jnp.float32),
        ],
    )(z_flat, codebook)


def _sc_gather_body(cb_hbm, idx_hbm, zq_hbm, idx_v, rows_v, sem):
    wid = lax.axis_index("s") * NC + lax.axis_index("c")
    base = wid * RPW
    b = wid // (1024 // RPW)
    t0 = (wid % (1024 // RPW)) * RPW
    for c in range(NCH):
        pltpu.sync_copy(idx_hbm.at[pl.ds(base + c * CH, CH)], idx_v.at[c])
    copies = [pltpu.async_copy(cb_hbm.at[idx_v.at[c]], rows_v.at[c], sem)
              for c in range(NCH)]
    for c in range(NCH):
        copies[c].wait()
        pltpu.sync_copy(rows_v.at[c], zq_hbm.at[b, pl.ds(t0 + c * CH, CH)])


def _sc_gather(codebook, idx_flat):
    mesh = plsc.VectorSubcoreMesh(core_axis_name="c", subcore_axis_name="s")
    f = functools.partial(
        pl.kernel,
        mesh=mesh,
        out_type=jax.ShapeDtypeStruct((8, 1024, D), jnp.float32),
        scratch_types=[
            pltpu.VMEM((NCH, CH), jnp.int32),
            pltpu.VMEM((NCH, CH, D), jnp.float32),
            pltpu.SemaphoreType.DMA,
        ],
        compiler_params=pltpu.CompilerParams(
            needs_layout_passes=False, use_tc_tiling_on_sc=False),
    )(_sc_gather_body)
    return f(codebook, idx_flat)


def kernel(z, codebook):
    z_flat = z.reshape(-1, D)
    idx_flat, loss2, perp2 = _tc_main(z_flat, codebook)
    z_q_st = _sc_gather(codebook, idx_flat)
    indices = idx_flat.reshape(z.shape[:-1])
    return (loss2[0, 0], z_q_st, perp2[0, 0], indices)


# speedup vs baseline: 1.4516x; 1.0654x over previous
import functools

import jax
import jax.numpy as jnp
from jax import lax
from jax.experimental import pallas as pl
from jax.experimental.pallas import tpu as pltpu
from jax.experimental.pallas import tpu_sc as plsc

K = 1024
D = 64
BETA = 0.25
N = 8192
BLK = 4096
NBLK = N // BLK

NC = 2
NS = 16
NW = NC * NS
RPW = N // NW
CH = 128
NCH = RPW // CH


def _main_body(z_ref, cb_ref, idx_ref, loss_ref, perp_ref,
               c2_s, hist_s, lsum_s):
    i = pl.program_id(0)
    cb = cb_ref[...]

    @pl.when(i == 0)
    def _init():
        c2 = jnp.sum(cb * cb, axis=1)
        c2_s[...] = c2[None, :]
        hist_s[...] = jnp.zeros((1, K), jnp.float32)
        lsum_s[0] = 0.0

    zt = z_ref[...]
    z = jnp.concatenate(
        [lax.transpose(zt[b], (1, 0)) for b in range(8)], axis=0)
    s = lax.dot_general(z, cb, (((1,), (1,)), ((), ())),
                        preferred_element_type=jnp.float32)
    z2 = jnp.sum(z * z, axis=1, keepdims=True)
    d = z2 + c2_s[...] - 2.0 * s
    mind = jnp.min(d, axis=1)
    iota1 = lax.broadcasted_iota(jnp.int32, (1, K), 1)
    idx = jnp.min(jnp.where(d == mind[:, None], iota1, K), axis=1)
    idx_ref[...] = idx.reshape(8, BLK // 8)
    lsum_s[0] += jnp.sum(mind)
    eqf = (idx[:, None] == iota1).astype(jnp.float32)
    ones_row = jnp.ones((1, BLK), jnp.float32)
    counts_blk = lax.dot_general(ones_row, eqf, (((1,), (0,)), ((), ())),
                                 preferred_element_type=jnp.float32)
    hist_s[...] += counts_blk

    @pl.when(i == NBLK - 1)
    def _fin():
        m = lsum_s[0] / float(N * D)
        counts = hist_s[...]
        e = counts * (1.0 / float(N))
        ent = jnp.sum(e * jnp.log(e + 1e-10))
        loss_ref[...] = jnp.full((1, 1), m + BETA * m, jnp.float32)
        perp_ref[...] = jnp.full((1, 1), jnp.exp(-ent), jnp.float32)


def _tc_main(z_t, codebook):
    tok = BLK // 8
    return pl.pallas_call(
        _main_body,
        grid=(NBLK,),
        in_specs=[
            pl.BlockSpec((8, D, tok), lambda i: (0, 0, i)),
            pl.BlockSpec((K, D), lambda i: (0, 0)),
        ],
        out_specs=[
            pl.BlockSpec((8, tok), lambda i: (0, i)),
            pl.BlockSpec((1, 1), lambda i: (0, 0)),
            pl.BlockSpec((1, 1), lambda i: (0, 0)),
        ],
        out_shape=[
            jax.ShapeDtypeStruct((8, 1024), jnp.int32),
            jax.ShapeDtypeStruct((1, 1), jnp.float32),
            jax.ShapeDtypeStruct((1, 1), jnp.float32),
        ],
        scratch_shapes=[
            pltpu.VMEM((1, K), jnp.float32),
            pltpu.VMEM((1, K), jnp.float32),
            pltpu.SMEM((1,), jnp.float32),
        ],
    )(z_t, codebook)


def _sc_gather_body(cb_hbm, idx_hbm, zq_hbm, idx_v, rows_v, sem):
    wid = lax.axis_index("s") * NC + lax.axis_index("c")
    base = wid * RPW
    b = wid // (1024 // RPW)
    t0 = (wid % (1024 // RPW)) * RPW
    del base
    for c in range(NCH):
        pltpu.sync_copy(idx_hbm.at[b, pl.ds(t0 + c * CH, CH)], idx_v.at[c])
    copies = [pltpu.async_copy(cb_hbm.at[idx_v.at[c]], rows_v.at[c], sem)
              for c in range(NCH)]
    for c in range(NCH):
        copies[c].wait()
        pltpu.sync_copy(rows_v.at[c], zq_hbm.at[b, pl.ds(t0 + c * CH, CH)])


def _sc_gather(codebook, idx2):
    mesh = plsc.VectorSubcoreMesh(core_axis_name="c", subcore_axis_name="s")
    f = functools.partial(
        pl.kernel,
        mesh=mesh,
        out_type=jax.ShapeDtypeStruct((8, 1024, D), jnp.float32),
        scratch_types=[
            pltpu.VMEM((NCH, CH), jnp.int32),
            pltpu.VMEM((NCH, CH, D), jnp.float32),
            pltpu.SemaphoreType.DMA,
        ],
        compiler_params=pltpu.CompilerParams(
            needs_layout_passes=False, use_tc_tiling_on_sc=False),
    )(_sc_gather_body)
    return f(codebook, idx2)


def kernel(z, codebook):
    z_t = jnp.transpose(z, (0, 2, 1))
    indices, loss2, perp2 = _tc_main(z_t, codebook)
    z_q_st = _sc_gather(codebook, indices)
    return (loss2[0, 0], z_q_st, perp2[0, 0], indices)
